# baseline (speedup 1.0000x reference)
import functools

import jax
import jax.numpy as jnp
from jax import lax
from jax.experimental import pallas as pl
from jax.experimental.pallas import tpu as pltpu
from jax.experimental.pallas import tpu_sc as plsc

NUM_INPUTS = 26
STATE_SIZE = 32
CARDINALITY = 100000
BATCH = 16384

NC = 2
NS = 16
NW = NC * NS
COLS = NUM_INPUTS * STATE_SIZE
CPW = COLS // NW
CHUNK = 4096
NCHUNK = BATCH // CHUNK
L = 16

def _make_kernel():
    mesh = plsc.VectorSubcoreMesh(core_axis_name="c", subcore_axis_name="s")

    @functools.partial(
        pl.kernel,
        mesh=mesh,
        out_type=jax.ShapeDtypeStruct((NUM_INPUTS, STATE_SIZE, BATCH), jnp.float32),
        scratch_types=[
            pltpu.VMEM((CARDINALITY,), jnp.float32),
            pltpu.VMEM((BATCH,), jnp.int32),
            pltpu.VMEM((2, CHUNK), jnp.float32),
            pltpu.SemaphoreType.DMA,
            pltpu.SemaphoreType.DMA,
            pltpu.SemaphoreType.DMA,
        ],
        compiler_params=pltpu.CompilerParams(needs_layout_passes=False),
    )
    def col_kernel(xt_hbm, tabt_hbm, out_hbm, col_v, idx_v, res_v, sem_c, sem_i, sem_o):
        wid = lax.axis_index("s") * NC + lax.axis_index("c")

        def fire_col(t, c):
            pltpu.async_copy(tabt_hbm.at[t, c], col_v, sem_c)

        def drain_col(t, c):
            pltpu.make_async_copy(tabt_hbm.at[t, c], col_v, sem_c).wait()

        def write_res(t, c, j, buf):
            pltpu.async_copy(
                res_v.at[buf], out_hbm.at[t, c, pl.ds(j * CHUNK, CHUNK)], sem_o
            )

        def wait_res(t, c, j, buf):
            pltpu.make_async_copy(
                res_v.at[buf], out_hbm.at[t, c, pl.ds(j * CHUNK, CHUNK)], sem_o
            ).wait()

        def do_col(k, _):
            tau = wid * CPW + k
            t = lax.div(tau, STATE_SIZE)
            c = lax.rem(tau, STATE_SIZE)
            fire_col(t, c)

            new_t = jnp.logical_or(k == 0, c == 0)

            @pl.when(new_t)
            def _():
                pltpu.async_copy(xt_hbm.at[t], idx_v, sem_i)
                pltpu.make_async_copy(xt_hbm.at[t], idx_v, sem_i).wait()

            drain_col(t, c)

            def do_chunk(j, _):
                buf = lax.rem(j, 2)

                @pl.when(j >= 2)
                def _():
                    wait_res(t, c, j - 2, buf)

                @plsc.parallel_loop(0, CHUNK, step=L, unroll=16)
                def _(i):
                    idx = idx_v[pl.ds(j * CHUNK + i, L)]
                    res_v[buf, pl.ds(i, L)] = plsc.load_gather(col_v, [idx])
                write_res(t, c, j, buf)
                return ()

            lax.fori_loop(0, NCHUNK, do_chunk, (), unroll=False)
            for j in (NCHUNK - 2, NCHUNK - 1):
                wait_res(t, c, j, j % 2)
            return ()

        lax.fori_loop(0, CPW, do_col, (), unroll=False)

    return col_kernel


_KERNEL = _make_kernel()


@jax.jit
def kernel(x, tables):
    xt = x.T.astype(jnp.int32)
    tabt = tables.transpose(0, 2, 1)
    out = _KERNEL(xt, tabt)
    return out.transpose(0, 2, 1)
